# baseline (speedup 1.0000x reference)
import jax
import jax.numpy as jnp
from jax.experimental import pallas as pl

_N = 100000
_NG = 256


def _classifier_body(m_ref, W1, b1, W2, b2, W3, b3, W4, b4, Wc, bc, hg_ref, lg_ref):
    g = jax.nn.relu(W1[0, :] + b1[:])
    g = jax.nn.relu(jnp.dot(g, W2[...]) + b2[:])
    g = jax.nn.relu(jnp.dot(g, W3[...]) + b3[:])
    g = jax.nn.relu(jnp.dot(g, W4[...]) + b4[:])
    wc = jnp.dot(g, Wc[...])
    m = m_ref[:]
    bcv = bc[:]
    hg_ref[...] = m[:, None] * g[None, :]
    lg_ref[...] = m[:, None] * wc[None, :] + bcv[None, :]


def kernel(edge_index, graph_ids, W1, b1, W2, b2, W3, b3, W4, b4, Wc, bc):
    src = edge_index[0]
    dst = edge_index[1]
    ones_e = jnp.ones((edge_index.shape[1],), jnp.float32)
    in_deg = jax.ops.segment_sum(ones_e, dst, num_segments=_N)
    out_deg = jax.ops.segment_sum(ones_e, src, num_segments=_N)
    norm_src = jnp.maximum(out_deg, 1.0) ** -0.5
    norm_dst = jnp.maximum(in_deg, 1.0) ** -0.5
    s = in_deg
    for _ in range(4):
        s = norm_dst * jax.ops.segment_sum((norm_src * s)[src], dst, num_segments=_N)
    counts = jnp.maximum(
        jax.ops.segment_sum(jnp.ones((_N,), jnp.float32), graph_ids, num_segments=_NG), 1.0)
    m = jax.ops.segment_sum(s, graph_ids, num_segments=_NG) / counts

    hg, logits = pl.pallas_call(
        _classifier_body,
        out_shape=(
            jax.ShapeDtypeStruct((_NG, 32), jnp.float32),
            jax.ShapeDtypeStruct((_NG, 10), jnp.float32),
        ),
    )(m, W1, b1, W2, b2, W3, b3, W4, b4, Wc, bc)
    return (hg, logits)


# speedup vs baseline: 57.7516x; 57.7516x over previous
import functools

import jax
import jax.numpy as jnp
from jax import lax
from jax.experimental import pallas as pl
from jax.experimental.pallas import tpu as pltpu
from jax.experimental.pallas import tpu_sc as plsc

N = 100000
E = 1600000
NG = 256
H = 32
NC = 2
NS = 16
NW = NC * NS
L = 16

Nn = 102400
TS = Nn // NW
WS = Nn // NS
GROWS = Nn // NW // 128
BK = 8
NBLK = 392
TOUT = NBLK // BK
Ep = NW * NBLK * 128
NGp = 384

_mesh = plsc.VectorSubcoreMesh(core_axis_name="c", subcore_axis_name="s")
_f32 = jnp.float32
_i32 = jnp.int32


def _wid():
    c = lax.axis_index("c")
    s = lax.axis_index("s")
    return c, s, c * NS + s


def _deg_body(srcR, dstR, gidR, zeros_h, zg_h, ones_h, odp, idp, gcp,
              acc_o, acc_i, gacc, sbuf, dbuf, obuf, gbuf, sem1, sem2):
    c, s, w = _wid()
    pltpu.sync_copy(zeros_h, acc_o.at[pl.ds(s * WS, WS)])
    pltpu.sync_copy(zeros_h, acc_i.at[pl.ds(s * WS, WS)])

    @pl.when(s == 0)
    def _():
        pltpu.sync_copy(zg_h, gacc)

    pltpu.sync_copy(ones_h, obuf)
    plsc.subcore_barrier()

    def t_body(t, carry):
        pltpu.sync_copy(srcR.at[w, pl.ds(t * BK, BK)], sbuf)
        pltpu.sync_copy(dstR.at[w, pl.ds(t * BK, BK)], dbuf)
        ds_ = [pltpu.async_copy(obuf.at[j], acc_o.at[sbuf.at[j]], sem1, add=True)
               for j in range(BK)]
        dd_ = [pltpu.async_copy(obuf.at[j], acc_i.at[dbuf.at[j]], sem2, add=True)
               for j in range(BK)]
        for d in ds_ + dd_:
            d.wait()
        return carry

    lax.fori_loop(0, TOUT, t_body, 0)

    pltpu.sync_copy(gidR.at[w], gbuf)
    dg = [pltpu.async_copy(obuf.at[j % BK], gacc.at[gbuf.at[j]], sem1, add=True)
          for j in range(GROWS)]
    for d in dg:
        d.wait()

    plsc.subcore_barrier()
    pltpu.sync_copy(acc_o.at[pl.ds(s * WS, WS)], odp.at[c, pl.ds(s * WS, WS)])
    pltpu.sync_copy(acc_i.at[pl.ds(s * WS, WS)], idp.at[c, pl.ds(s * WS, WS)])

    @pl.when(s == 0)
    def _():
        pltpu.sync_copy(gacc, gcp.at[c])


def _comb0_body(odp, idp, nsnd_ref, nd_ref, q_ref):
    od = odp[0, :] + odp[1, :]
    ig = idp[0, :] + idp[1, :]
    ns = lax.rsqrt(jnp.maximum(od, 1.0))
    nd = lax.rsqrt(jnp.maximum(ig, 1.0))
    nsnd_ref[...] = ns * nd
    nd_ref[...] = nd
    q_ref[...] = ns * ig


def _pass_body(srcR, dstR, q_h, zeros_h, part, acc, sbuf, dbuf, vbuf, gsem, ssem):
    c, s, w = _wid()
    pltpu.sync_copy(zeros_h, acc.at[pl.ds(s * WS, WS)])
    plsc.subcore_barrier()

    def t_body(t, carry):
        pltpu.sync_copy(srcR.at[w, pl.ds(t * BK, BK)], sbuf)
        pltpu.sync_copy(dstR.at[w, pl.ds(t * BK, BK)], dbuf)
        gd = [pltpu.async_copy(q_h.at[sbuf.at[j]], vbuf.at[j], gsem)
              for j in range(BK)]
        for d in gd:
            d.wait()
        sd = [pltpu.async_copy(vbuf.at[j], acc.at[dbuf.at[j]], ssem, add=True)
              for j in range(BK)]
        for d in sd:
            d.wait()
        return carry

    lax.fori_loop(0, TOUT, t_body, 0)
    plsc.subcore_barrier()
    pltpu.sync_copy(acc.at[pl.ds(s * WS, WS)], part.at[c, pl.ds(s * WS, WS)])


def _combk_body(part, nsnd, q_ref):
    q_ref[...] = nsnd[...] * (part[0, :] + part[1, :])


def _pool_body(part, nd_h, gidR, zg_h, gpool, gacc, pa, pb, ndb, sb, gbuf, sem):
    c, s, w = _wid()
    base = w * TS
    pltpu.sync_copy(part.at[0, pl.ds(base, TS)], pa)
    pltpu.sync_copy(part.at[1, pl.ds(base, TS)], pb)
    pltpu.sync_copy(nd_h.at[pl.ds(base, TS)], ndb)
    pltpu.sync_copy(gidR.at[w], gbuf)

    @pl.when(s == 0)
    def _():
        pltpu.sync_copy(zg_h, gacc)

    def body(i, carry):
        sl = pl.ds(i * L, L)
        sb[sl] = ndb[sl] * (pa[sl] + pb[sl])
        return carry

    lax.fori_loop(0, TS // L, body, 0)
    plsc.subcore_barrier()
    dg = [pltpu.async_copy(sb.at[pl.ds(j * 128, 128)], gacc.at[gbuf.at[j]],
                           sem, add=True)
          for j in range(GROWS)]
    for d in dg:
        d.wait()
    plsc.subcore_barrier()

    @pl.when(s == 0)
    def _():
        pltpu.sync_copy(gacc, gpool.at[c])


def _classifier_body(gpool, gcp, W1, b1, W2, b2, W3, b3, W4, b4, Wc, bc,
                     hg_ref, lg_ref):
    cnt = jnp.maximum(gcp[0, :NG] + gcp[1, :NG], 1.0)
    m = (gpool[0, :NG] + gpool[1, :NG]) / cnt
    g = jax.nn.relu(W1[0, :] + b1[:])
    g = jax.nn.relu(jnp.dot(g, W2[...]) + b2[:])
    g = jax.nn.relu(jnp.dot(g, W3[...]) + b3[:])
    g = jax.nn.relu(jnp.dot(g, W4[...]) + b4[:])
    wc = jnp.dot(g, Wc[...])
    bcv = bc[:]
    hg_ref[...] = m[:, None] * g[None, :]
    lg_ref[...] = m[:, None] * wc[None, :] + bcv[None, :]


_deg_k = functools.partial(
    pl.kernel, _deg_body, mesh=_mesh,
    out_type=(jax.ShapeDtypeStruct((NC, Nn), _f32),
              jax.ShapeDtypeStruct((NC, Nn), _f32),
              jax.ShapeDtypeStruct((NC, NGp), _f32)),
    scratch_types=[pltpu.VMEM_SHARED((Nn,), _f32),
                   pltpu.VMEM_SHARED((Nn,), _f32),
                   pltpu.VMEM_SHARED((NGp,), _f32),
                   pltpu.VMEM((BK, 128), _i32),
                   pltpu.VMEM((BK, 128), _i32),
                   pltpu.VMEM((BK, 128), _f32),
                   pltpu.VMEM((GROWS, 128), _i32),
                   pltpu.SemaphoreType.DMA,
                   pltpu.SemaphoreType.DMA])

def _comb0_tc(odp, idp):
    return pl.pallas_call(
        _comb0_body,
        out_shape=(jax.ShapeDtypeStruct((Nn,), _f32),
                   jax.ShapeDtypeStruct((Nn,), _f32),
                   jax.ShapeDtypeStruct((Nn,), _f32)),
    )(odp, idp)


def _combk_tc(part, nsnd):
    return pl.pallas_call(
        _combk_body,
        out_shape=jax.ShapeDtypeStruct((Nn,), _f32),
    )(part, nsnd)

_pass_k = functools.partial(
    pl.kernel, _pass_body, mesh=_mesh,
    out_type=jax.ShapeDtypeStruct((NC, Nn), _f32),
    scratch_types=[pltpu.VMEM_SHARED((Nn,), _f32),
                   pltpu.VMEM((BK, 128), _i32),
                   pltpu.VMEM((BK, 128), _i32),
                   pltpu.VMEM((BK, 128), _f32),
                   pltpu.SemaphoreType.DMA,
                   pltpu.SemaphoreType.DMA])

_pool_k = functools.partial(
    pl.kernel, _pool_body, mesh=_mesh,
    out_type=jax.ShapeDtypeStruct((NC, NGp), _f32),
    scratch_types=[pltpu.VMEM_SHARED((NGp,), _f32),
                   pltpu.VMEM((TS,), _f32),
                   pltpu.VMEM((TS,), _f32),
                   pltpu.VMEM((TS,), _f32),
                   pltpu.VMEM((TS,), _f32),
                   pltpu.VMEM((GROWS, 128), _i32),
                   pltpu.SemaphoreType.DMA])


def kernel(edge_index, graph_ids, W1, b1, W2, b2, W3, b3, W4, b4, Wc, bc):
    srcR = jnp.pad(edge_index[0], (0, Ep - E), constant_values=N).reshape(NW, NBLK, 128)
    dstR = jnp.pad(edge_index[1], (0, Ep - E), constant_values=N).reshape(NW, NBLK, 128)
    gidR = jnp.pad(graph_ids, (0, Nn - N), constant_values=NG).reshape(NW, GROWS, 128)
    zeros_h = jnp.zeros((WS,), _f32)
    zg_h = jnp.zeros((NGp,), _f32)
    ones_h = jnp.ones((BK, 128), _f32)

    odp, idp, gcp = _deg_k()(srcR, dstR, gidR, zeros_h, zg_h, ones_h)
    nsnd_h, nd_h, q = _comb0_tc(odp, idp)
    for _ in range(3):
        part = _pass_k()(srcR, dstR, q, zeros_h)
        q = _combk_tc(part, nsnd_h)
    part = _pass_k()(srcR, dstR, q, zeros_h)
    gpool = _pool_k()(part, nd_h, gidR, zg_h)

    hg, logits = pl.pallas_call(
        _classifier_body,
        out_shape=(jax.ShapeDtypeStruct((NG, H), _f32),
                   jax.ShapeDtypeStruct((NG, 10), _f32)),
    )(gpool, gcp, W1, b1, W2, b2, W3, b3, W4, b4, Wc, bc)
    return (hg, logits)


# speedup vs baseline: 57.8208x; 1.0012x over previous
import functools

import jax
import jax.numpy as jnp
from jax import lax
from jax.experimental import pallas as pl
from jax.experimental.pallas import tpu as pltpu
from jax.experimental.pallas import tpu_sc as plsc

N = 100000
E = 1600000
NG = 256
H = 32
NC = 2
NS = 16
NW = NC * NS
L = 16

Nn = 102400
TS = Nn // NW
WS = Nn // NS
GROWS = Nn // NW // 128
BK = 8
NBLK = 392
TOUT = NBLK // BK
BKF = BK * 128
Ep = NW * NBLK * 128
NGp = 384

_mesh = plsc.VectorSubcoreMesh(core_axis_name="c", subcore_axis_name="s")
_f32 = jnp.float32
_i32 = jnp.int32


def _wid():
    c = lax.axis_index("c")
    s = lax.axis_index("s")
    return c, s, c * NS + s


def _deg_body(srcR, dstR, gidR, zeros_h, zg_h, ones_h, odp, idp, gcp,
              acc_o, acc_i, gacc, sbuf, dbuf, obuf, gbuf, sem1, sem2):
    c, s, w = _wid()
    pltpu.sync_copy(zeros_h, acc_o.at[pl.ds(s * WS, WS)])
    pltpu.sync_copy(zeros_h, acc_i.at[pl.ds(s * WS, WS)])

    @pl.when(s == 0)
    def _():
        pltpu.sync_copy(zg_h, gacc)

    pltpu.sync_copy(ones_h, obuf)
    plsc.subcore_barrier()

    def t_body(t, carry):
        pltpu.sync_copy(srcR.at[w, pl.ds(t * BK, BK)], sbuf)
        pltpu.sync_copy(dstR.at[w, pl.ds(t * BK, BK)], dbuf)
        ds_ = [pltpu.async_copy(obuf.at[j], acc_o.at[sbuf.at[j]], sem1, add=True)
               for j in range(BK)]
        dd_ = [pltpu.async_copy(obuf.at[j], acc_i.at[dbuf.at[j]], sem2, add=True)
               for j in range(BK)]
        for d in ds_ + dd_:
            d.wait()
        return carry

    lax.fori_loop(0, TOUT, t_body, 0)

    pltpu.sync_copy(gidR.at[w], gbuf)
    dg = [pltpu.async_copy(obuf.at[j % BK], gacc.at[gbuf.at[j]], sem1, add=True)
          for j in range(GROWS)]
    for d in dg:
        d.wait()

    plsc.subcore_barrier()
    pltpu.sync_copy(acc_o.at[pl.ds(s * WS, WS)], odp.at[c, pl.ds(s * WS, WS)])
    pltpu.sync_copy(acc_i.at[pl.ds(s * WS, WS)], idp.at[c, pl.ds(s * WS, WS)])

    @pl.when(s == 0)
    def _():
        pltpu.sync_copy(gacc, gcp.at[c])


def _comb0_body(odp, idp, nsnd_ref, nd_ref, q_ref):
    od = odp[0, :] + odp[1, :]
    ig = idp[0, :] + idp[1, :]
    ns = lax.rsqrt(jnp.maximum(od, 1.0))
    nd = lax.rsqrt(jnp.maximum(ig, 1.0))
    nsnd_ref[...] = ns * nd
    nd_ref[...] = nd
    q_ref[...] = ns * ig


def _pass_body(srcR, dstR, q_h, zeros_h, part, acc, sbuf, dbuf, vbuf, gsem, ssem):
    c, s, w = _wid()
    pltpu.sync_copy(zeros_h, acc.at[pl.ds(s * WS, WS)])
    plsc.subcore_barrier()

    def t_body(t, carry):
        pltpu.sync_copy(srcR.at[w, t], sbuf)
        pltpu.sync_copy(dstR.at[w, t], dbuf)
        pltpu.async_copy(q_h.at[sbuf], vbuf, gsem).wait()
        pltpu.async_copy(vbuf, acc.at[dbuf], ssem, add=True).wait()
        return carry

    lax.fori_loop(0, TOUT, t_body, 0)
    plsc.subcore_barrier()
    pltpu.sync_copy(acc.at[pl.ds(s * WS, WS)], part.at[c, pl.ds(s * WS, WS)])


def _combk_body(part, nsnd, q_ref):
    q_ref[...] = nsnd[...] * (part[0, :] + part[1, :])


def _pool_body(part, nd_h, gidR, zg_h, gpool, gacc, pa, pb, ndb, sb, gbuf, sem):
    c, s, w = _wid()
    base = w * TS
    pltpu.sync_copy(part.at[0, pl.ds(base, TS)], pa)
    pltpu.sync_copy(part.at[1, pl.ds(base, TS)], pb)
    pltpu.sync_copy(nd_h.at[pl.ds(base, TS)], ndb)
    pltpu.sync_copy(gidR.at[w], gbuf)

    @pl.when(s == 0)
    def _():
        pltpu.sync_copy(zg_h, gacc)

    def body(i, carry):
        sl = pl.ds(i * L, L)
        sb[sl] = ndb[sl] * (pa[sl] + pb[sl])
        return carry

    lax.fori_loop(0, TS // L, body, 0)
    plsc.subcore_barrier()
    dg = [pltpu.async_copy(sb.at[pl.ds(j * 128, 128)], gacc.at[gbuf.at[j]],
                           sem, add=True)
          for j in range(GROWS)]
    for d in dg:
        d.wait()
    plsc.subcore_barrier()

    @pl.when(s == 0)
    def _():
        pltpu.sync_copy(gacc, gpool.at[c])


def _classifier_body(gpool, gcp, W1, b1, W2, b2, W3, b3, W4, b4, Wc, bc,
                     hg_ref, lg_ref):
    cnt = jnp.maximum(gcp[0, :NG] + gcp[1, :NG], 1.0)
    m = (gpool[0, :NG] + gpool[1, :NG]) / cnt
    g = jax.nn.relu(W1[0, :] + b1[:])
    g = jax.nn.relu(jnp.dot(g, W2[...]) + b2[:])
    g = jax.nn.relu(jnp.dot(g, W3[...]) + b3[:])
    g = jax.nn.relu(jnp.dot(g, W4[...]) + b4[:])
    wc = jnp.dot(g, Wc[...])
    bcv = bc[:]
    hg_ref[...] = m[:, None] * g[None, :]
    lg_ref[...] = m[:, None] * wc[None, :] + bcv[None, :]


_deg_k = functools.partial(
    pl.kernel, _deg_body, mesh=_mesh,
    out_type=(jax.ShapeDtypeStruct((NC, Nn), _f32),
              jax.ShapeDtypeStruct((NC, Nn), _f32),
              jax.ShapeDtypeStruct((NC, NGp), _f32)),
    scratch_types=[pltpu.VMEM_SHARED((Nn,), _f32),
                   pltpu.VMEM_SHARED((Nn,), _f32),
                   pltpu.VMEM_SHARED((NGp,), _f32),
                   pltpu.VMEM((BK, 128), _i32),
                   pltpu.VMEM((BK, 128), _i32),
                   pltpu.VMEM((BK, 128), _f32),
                   pltpu.VMEM((GROWS, 128), _i32),
                   pltpu.SemaphoreType.DMA,
                   pltpu.SemaphoreType.DMA])

def _comb0_tc(odp, idp):
    return pl.pallas_call(
        _comb0_body,
        out_shape=(jax.ShapeDtypeStruct((Nn,), _f32),
                   jax.ShapeDtypeStruct((Nn,), _f32),
                   jax.ShapeDtypeStruct((Nn,), _f32)),
    )(odp, idp)


def _combk_tc(part, nsnd):
    return pl.pallas_call(
        _combk_body,
        out_shape=jax.ShapeDtypeStruct((Nn,), _f32),
    )(part, nsnd)

_pass_k = functools.partial(
    pl.kernel, _pass_body, mesh=_mesh,
    out_type=jax.ShapeDtypeStruct((NC, Nn), _f32),
    scratch_types=[pltpu.VMEM_SHARED((Nn,), _f32),
                   pltpu.VMEM((BKF,), _i32),
                   pltpu.VMEM((BKF,), _i32),
                   pltpu.VMEM((BKF,), _f32),
                   pltpu.SemaphoreType.DMA,
                   pltpu.SemaphoreType.DMA])

_pool_k = functools.partial(
    pl.kernel, _pool_body, mesh=_mesh,
    out_type=jax.ShapeDtypeStruct((NC, NGp), _f32),
    scratch_types=[pltpu.VMEM_SHARED((NGp,), _f32),
                   pltpu.VMEM((TS,), _f32),
                   pltpu.VMEM((TS,), _f32),
                   pltpu.VMEM((TS,), _f32),
                   pltpu.VMEM((TS,), _f32),
                   pltpu.VMEM((GROWS, 128), _i32),
                   pltpu.SemaphoreType.DMA])


def kernel(edge_index, graph_ids, W1, b1, W2, b2, W3, b3, W4, b4, Wc, bc):
    srcR = jnp.pad(edge_index[0], (0, Ep - E), constant_values=N).reshape(NW, NBLK, 128)
    dstR = jnp.pad(edge_index[1], (0, Ep - E), constant_values=N).reshape(NW, NBLK, 128)
    gidR = jnp.pad(graph_ids, (0, Nn - N), constant_values=NG).reshape(NW, GROWS, 128)
    zeros_h = jnp.zeros((WS,), _f32)
    zg_h = jnp.zeros((NGp,), _f32)
    ones_h = jnp.ones((BK, 128), _f32)

    srcF = srcR.reshape(NW, TOUT, BKF)
    dstF = dstR.reshape(NW, TOUT, BKF)
    odp, idp, gcp = _deg_k()(srcR, dstR, gidR, zeros_h, zg_h, ones_h)
    nsnd_h, nd_h, q = _comb0_tc(odp, idp)
    for _ in range(3):
        part = _pass_k()(srcF, dstF, q, zeros_h)
        q = _combk_tc(part, nsnd_h)
    part = _pass_k()(srcF, dstF, q, zeros_h)
    gpool = _pool_k()(part, nd_h, gidR, zg_h)

    hg, logits = pl.pallas_call(
        _classifier_body,
        out_shape=(jax.ShapeDtypeStruct((NG, H), _f32),
                   jax.ShapeDtypeStruct((NG, 10), _f32)),
    )(gpool, gcp, W1, b1, W2, b2, W3, b3, W4, b4, Wc, bc)
    return (hg, logits)


# speedup vs baseline: 123.1107x; 2.1292x over previous
import functools

import jax
import jax.numpy as jnp
from jax import lax
from jax.experimental import pallas as pl
from jax.experimental.pallas import tpu as pltpu
from jax.experimental.pallas import tpu_sc as plsc

N = 100000
E = 1600000
NG = 256
H = 32
NC = 2
NS = 16
NW = NC * NS
L = 16

Nn = 102400
TS = Nn // NW
WS = Nn // NS
GROWS = Nn // NW // 128
BK = 8
NBLK = 392
TOUT = 28
BKF = NBLK * 128 // TOUT
Ep = NW * NBLK * 128
NGp = 384

_mesh = plsc.VectorSubcoreMesh(core_axis_name="c", subcore_axis_name="s")
_f32 = jnp.float32
_i32 = jnp.int32


def _wid():
    c = lax.axis_index("c")
    s = lax.axis_index("s")
    return c, s, c * NS + s


def _deg_body(srcR, dstR, gidR, zeros_h, zg_h, ones_h, odp, idp, gcp,
              acc_o, acc_i, gacc, sbuf, dbuf, obuf, gbuf, sem1, sem2):
    c, s, w = _wid()
    pltpu.sync_copy(zeros_h, acc_o.at[pl.ds(s * WS, WS)])
    pltpu.sync_copy(zeros_h, acc_i.at[pl.ds(s * WS, WS)])

    @pl.when(s == 0)
    def _():
        pltpu.sync_copy(zg_h, gacc)

    pltpu.sync_copy(ones_h, obuf)
    plsc.subcore_barrier()

    def t_body(t, carry):
        pltpu.sync_copy(srcR.at[w, pl.ds(t * BK, BK)], sbuf)
        pltpu.sync_copy(dstR.at[w, pl.ds(t * BK, BK)], dbuf)
        ds_ = [pltpu.async_copy(obuf.at[j], acc_o.at[sbuf.at[j]], sem1, add=True)
               for j in range(BK)]
        dd_ = [pltpu.async_copy(obuf.at[j], acc_i.at[dbuf.at[j]], sem2, add=True)
               for j in range(BK)]
        for d in ds_ + dd_:
            d.wait()
        return carry

    lax.fori_loop(0, TOUT, t_body, 0)

    pltpu.sync_copy(gidR.at[w], gbuf)
    dg = [pltpu.async_copy(obuf.at[j % BK], gacc.at[gbuf.at[j]], sem1, add=True)
          for j in range(GROWS)]
    for d in dg:
        d.wait()

    plsc.subcore_barrier()
    pltpu.sync_copy(acc_o.at[pl.ds(s * WS, WS)], odp.at[c, pl.ds(s * WS, WS)])
    pltpu.sync_copy(acc_i.at[pl.ds(s * WS, WS)], idp.at[c, pl.ds(s * WS, WS)])

    @pl.when(s == 0)
    def _():
        pltpu.sync_copy(gacc, gcp.at[c])


def _comb0_body(odp, idp, nsnd_ref, nd_ref, q_ref):
    od = odp[0, :] + odp[1, :]
    ig = idp[0, :] + idp[1, :]
    ns = lax.rsqrt(jnp.maximum(od, 1.0))
    nd = lax.rsqrt(jnp.maximum(ig, 1.0))
    nsnd_ref[...] = ns * nd
    nd_ref[...] = nd
    q_ref[...] = ns * ig


def _pass_body(srcF, dstF, q_h, zeros_h, part, acc, qtile,
               sbuf0, dbuf0, vbuf0, sbuf1, dbuf1, vbuf1, ssem):
    c, s, w = _wid()
    pltpu.sync_copy(zeros_h, acc.at[pl.ds(s * WS, WS)])
    pltpu.sync_copy(q_h, qtile)
    plsc.subcore_barrier()

    def gather_batch(sb, vb):
        def gb(i, carry):
            idx = sb[pl.ds(i * L, L)]
            vb[pl.ds(i * L, L)] = plsc.load_gather(qtile, [idx])
            return carry
        lax.fori_loop(0, BKF // L, gb, 0)

    def body(p, carry):
        for h, (sb, db, vb) in enumerate(((sbuf0, dbuf0, vbuf0),
                                          (sbuf1, dbuf1, vbuf1))):
            t = 2 * p + h

            @pl.when(p > 0)
            def _():
                pltpu.make_async_copy(q_h.at[pl.ds(0, BKF)], vb, ssem).wait()

            pltpu.sync_copy(srcF.at[w, t], sb)
            pltpu.sync_copy(dstF.at[w, t], db)
            gather_batch(sb, vb)
            pltpu.async_copy(vb, acc.at[db], ssem, add=True)
        return carry

    lax.fori_loop(0, TOUT // 2, body, 0)
    pltpu.make_async_copy(q_h.at[pl.ds(0, BKF)], vbuf0, ssem).wait()
    pltpu.make_async_copy(q_h.at[pl.ds(0, BKF)], vbuf1, ssem).wait()
    plsc.subcore_barrier()
    pltpu.sync_copy(acc.at[pl.ds(s * WS, WS)], part.at[c, pl.ds(s * WS, WS)])


def _combk_body(part, nsnd, q_ref):
    q_ref[...] = nsnd[...] * (part[0, :] + part[1, :])


def _pool_body(part, nd_h, gidR, zg_h, gpool, gacc, pa, pb, ndb, sb, gbuf, sem):
    c, s, w = _wid()
    base = w * TS
    pltpu.sync_copy(part.at[0, pl.ds(base, TS)], pa)
    pltpu.sync_copy(part.at[1, pl.ds(base, TS)], pb)
    pltpu.sync_copy(nd_h.at[pl.ds(base, TS)], ndb)
    pltpu.sync_copy(gidR.at[w], gbuf)

    @pl.when(s == 0)
    def _():
        pltpu.sync_copy(zg_h, gacc)

    def body(i, carry):
        sl = pl.ds(i * L, L)
        sb[sl] = ndb[sl] * (pa[sl] + pb[sl])
        return carry

    lax.fori_loop(0, TS // L, body, 0)
    plsc.subcore_barrier()
    dg = [pltpu.async_copy(sb.at[pl.ds(j * 128, 128)], gacc.at[gbuf.at[j]],
                           sem, add=True)
          for j in range(GROWS)]
    for d in dg:
        d.wait()
    plsc.subcore_barrier()

    @pl.when(s == 0)
    def _():
        pltpu.sync_copy(gacc, gpool.at[c])


def _classifier_body(gpool, gcp, W1, b1, W2, b2, W3, b3, W4, b4, Wc, bc,
                     hg_ref, lg_ref):
    cnt = jnp.maximum(gcp[0, :NG] + gcp[1, :NG], 1.0)
    m = (gpool[0, :NG] + gpool[1, :NG]) / cnt
    g = jax.nn.relu(W1[0, :] + b1[:])
    g = jax.nn.relu(jnp.dot(g, W2[...]) + b2[:])
    g = jax.nn.relu(jnp.dot(g, W3[...]) + b3[:])
    g = jax.nn.relu(jnp.dot(g, W4[...]) + b4[:])
    wc = jnp.dot(g, Wc[...])
    bcv = bc[:]
    hg_ref[...] = m[:, None] * g[None, :]
    lg_ref[...] = m[:, None] * wc[None, :] + bcv[None, :]


_deg_k = functools.partial(
    pl.kernel, _deg_body, mesh=_mesh,
    out_type=(jax.ShapeDtypeStruct((NC, Nn), _f32),
              jax.ShapeDtypeStruct((NC, Nn), _f32),
              jax.ShapeDtypeStruct((NC, NGp), _f32)),
    scratch_types=[pltpu.VMEM_SHARED((Nn,), _f32),
                   pltpu.VMEM_SHARED((Nn,), _f32),
                   pltpu.VMEM_SHARED((NGp,), _f32),
                   pltpu.VMEM((BK, 128), _i32),
                   pltpu.VMEM((BK, 128), _i32),
                   pltpu.VMEM((BK, 128), _f32),
                   pltpu.VMEM((GROWS, 128), _i32),
                   pltpu.SemaphoreType.DMA,
                   pltpu.SemaphoreType.DMA])

def _comb0_tc(odp, idp):
    return pl.pallas_call(
        _comb0_body,
        out_shape=(jax.ShapeDtypeStruct((Nn,), _f32),
                   jax.ShapeDtypeStruct((Nn,), _f32),
                   jax.ShapeDtypeStruct((Nn,), _f32)),
    )(odp, idp)


def _combk_tc(part, nsnd):
    return pl.pallas_call(
        _combk_body,
        out_shape=jax.ShapeDtypeStruct((Nn,), _f32),
    )(part, nsnd)

_pass_k = functools.partial(
    pl.kernel, _pass_body, mesh=_mesh,
    out_type=jax.ShapeDtypeStruct((NC, Nn), _f32),
    scratch_types=[pltpu.VMEM_SHARED((Nn,), _f32),
                   pltpu.VMEM((Nn,), _f32),
                   pltpu.VMEM((BKF,), _i32),
                   pltpu.VMEM((BKF,), _i32),
                   pltpu.VMEM((BKF,), _f32),
                   pltpu.VMEM((BKF,), _i32),
                   pltpu.VMEM((BKF,), _i32),
                   pltpu.VMEM((BKF,), _f32),
                   pltpu.SemaphoreType.DMA],
    compiler_params=pltpu.CompilerParams(needs_layout_passes=False))

_pool_k = functools.partial(
    pl.kernel, _pool_body, mesh=_mesh,
    out_type=jax.ShapeDtypeStruct((NC, NGp), _f32),
    scratch_types=[pltpu.VMEM_SHARED((NGp,), _f32),
                   pltpu.VMEM((TS,), _f32),
                   pltpu.VMEM((TS,), _f32),
                   pltpu.VMEM((TS,), _f32),
                   pltpu.VMEM((TS,), _f32),
                   pltpu.VMEM((GROWS, 128), _i32),
                   pltpu.SemaphoreType.DMA])


def kernel(edge_index, graph_ids, W1, b1, W2, b2, W3, b3, W4, b4, Wc, bc):
    srcR = jnp.pad(edge_index[0], (0, Ep - E), constant_values=N).reshape(NW, NBLK, 128)
    dstR = jnp.pad(edge_index[1], (0, Ep - E), constant_values=N).reshape(NW, NBLK, 128)
    gidR = jnp.pad(graph_ids, (0, Nn - N), constant_values=NG).reshape(NW, GROWS, 128)
    zeros_h = jnp.zeros((WS,), _f32)
    zg_h = jnp.zeros((NGp,), _f32)
    ones_h = jnp.ones((BK, 128), _f32)

    srcF = srcR.reshape(NW, TOUT, BKF)
    dstF = dstR.reshape(NW, TOUT, BKF)
    odp, idp, gcp = _deg_k()(srcR, dstR, gidR, zeros_h, zg_h, ones_h)
    nsnd_h, nd_h, q = _comb0_tc(odp, idp)
    for _ in range(3):
        part = _pass_k()(srcF, dstF, q, zeros_h)
        q = _combk_tc(part, nsnd_h)
    part = _pass_k()(srcF, dstF, q, zeros_h)
    gpool = _pool_k()(part, nd_h, gidR, zg_h)

    hg, logits = pl.pallas_call(
        _classifier_body,
        out_shape=(jax.ShapeDtypeStruct((NG, H), _f32),
                   jax.ShapeDtypeStruct((NG, 10), _f32)),
    )(gpool, gcp, W1, b1, W2, b2, W3, b3, W4, b4, Wc, bc)
    return (hg, logits)
